# baseline (speedup 1.0000x reference)
import functools

import jax
import jax.numpy as jnp
from jax import lax
from jax.experimental import pallas as pl
from jax.experimental.pallas import tpu as pltpu
from jax.experimental.pallas import tpu_sc as plsc

N = 50176
D = 96
HEADS = 4
HD = 24
HP = 32
DP = HEADS * HP
GS = 128
NG = N // GS
CG = 8
NCHUNK = NG // CG
NT = 8

NC = 2
NS = 16
NW = NC * NS
RPW = N // NW
CCH = 112
KCH = RPW // CCH

@functools.cache
def _sc_kernels():
    mesh = plsc.VectorSubcoreMesh(core_axis_name="c", subcore_axis_name="s")
    common = dict(
        mesh=mesh,
        compiler_params=pltpu.CompilerParams(use_tc_tiling_on_sc=False),
        out_type=jax.ShapeDtypeStruct((N, D), jnp.float32),
        scratch_types=[
            pltpu.VMEM((KCH, CCH), jnp.int32),
            pltpu.VMEM((CCH, D), jnp.float32),
            pltpu.SemaphoreType.DMA,
        ],
    )

    @functools.partial(pl.kernel, **common)
    def sc_gather(x_hbm, idx_hbm, out_hbm, idx_v, rows_v, sem):
        wid = lax.axis_index("s") * NC + lax.axis_index("c")
        base = wid * RPW
        pltpu.sync_copy(idx_hbm.at[wid], idx_v)
        for j in range(KCH):
            pltpu.async_copy(x_hbm.at[idx_v.at[j]], rows_v, sem).wait()
            pltpu.sync_copy(rows_v, out_hbm.at[pl.ds(base + j * CCH, CCH)])

    @functools.partial(pl.kernel, **common)
    def sc_scatter(o_hbm, idx_hbm, out_hbm, idx_v, rows_v, sem):
        wid = lax.axis_index("s") * NC + lax.axis_index("c")
        base = wid * RPW
        pltpu.sync_copy(idx_hbm.at[wid], idx_v)
        for j in range(KCH):
            pltpu.sync_copy(o_hbm.at[pl.ds(base + j * CCH, CCH)], rows_v)
            pltpu.async_copy(rows_v, out_hbm.at[idx_v.at[j]], sem).wait()

    return sc_gather, sc_scatter


def _attn_body(xa_ref, xb_ref, wqkv_ref, wp_ref, kg_ref, vg_ref, out_ref):
    f32 = jnp.float32
    xa = xa_ref[...]
    xb = xb_ref[...]
    wqkv = wqkv_ref[...]
    wp = wp_ref[...]
    kg = kg_ref[...]
    vg = vg_ref[...]

    def dot(a, b, dn):
        return lax.dot_general(a, b, dn, preferred_element_type=f32)

    M = CG * GS
    xc = jnp.concatenate([xa, xb], axis=0)
    qkv = dot(xc, wqkv, (((1,), (0,)), ((), ())))
    q = qkv[:M, :DP]
    kf = qkv[:, DP:2 * DP]
    vf = qkv[:, 2 * DP:]

    scale = HD ** -0.5
    sg = dot(q, kg, (((1,), (0,)), ((), ()))) * scale
    sg3 = sg.reshape(M, HEADS, NT)
    pg3 = jnp.exp(sg3 - jnp.max(sg3, axis=-1, keepdims=True))
    pg3 = pg3 / jnp.sum(pg3, axis=-1, keepdims=True)
    o2 = dot(pg3.reshape(M, HEADS * NT), vg,
             (((1,), (0,)), ((), ())))

    o_parts = []
    for h in range(HEADS):
        sl = slice(h * HP, (h + 1) * HP)
        qh = q[:, sl].reshape(CG, GS, HP)
        kh = jnp.concatenate(
            [kf[:M, sl].reshape(CG, GS, HP),
             kf[GS:, sl].reshape(CG, GS, HP)], axis=1)
        vh = jnp.concatenate(
            [vf[:M, sl].reshape(CG, GS, HP),
             vf[GS:, sl].reshape(CG, GS, HP)], axis=1)
        s = dot(qh, kh, (((2,), (2,)), ((0,), (0,)))) * scale
        m = jnp.max(s, axis=-1, keepdims=True)
        p = jnp.exp(s - m)
        l = jnp.sum(p, axis=-1, keepdims=True)
        o1 = dot(p, vh, (((2,), (1,)), ((0,), (0,)))) / l
        o_parts.append(o1.reshape(M, HP))
    o1c = jnp.concatenate(o_parts, axis=1)
    out_ref[...] = dot(o1c + o2, wp, (((1,), (0,)), ((), ())))


_attn = pl.pallas_call(
    _attn_body,
    grid=(NCHUNK,),
    in_specs=[
        pl.BlockSpec((CG * GS, D), lambda c: (c, 0)),
        pl.BlockSpec((GS, D), lambda c: (jnp.minimum(CG * c + CG, NG - 1), 0)),
        pl.BlockSpec((D, 3 * DP), lambda c: (0, 0)),
        pl.BlockSpec((DP, D), lambda c: (0, 0)),
        pl.BlockSpec((DP, HEADS * NT), lambda c: (0, 0)),
        pl.BlockSpec((HEADS * NT, DP), lambda c: (0, 0)),
    ],
    out_specs=pl.BlockSpec((CG * GS, D), lambda c: (c, 0)),
    out_shape=jax.ShapeDtypeStruct((N, D), jnp.float32),
)


def _pad_heads_rows(w):
    return jnp.pad(w.reshape(HEADS, HD, D),
                   ((0, 0), (0, HP - HD), (0, 0))).reshape(DP, D)


def kernel(normed_x, idx_last, k_global, v_global, Wq, Wk, Wv, Wproj):
    x = normed_x[0]
    perm = idx_last[0, :, 0].astype(jnp.int32)
    idx3 = perm.reshape(NW, KCH, CCH)

    wqkv = jnp.concatenate(
        [_pad_heads_rows(Wq).T, _pad_heads_rows(Wk).T,
         _pad_heads_rows(Wv).T], axis=1)
    wp = _pad_heads_rows(Wproj.T)
    eye = jnp.eye(HEADS, dtype=jnp.float32)
    kgp = jnp.pad(k_global, ((0, 0), (0, 0), (0, HP - HD)))
    vgp = jnp.pad(v_global, ((0, 0), (0, 0), (0, HP - HD)))
    kg = jnp.einsum('htd,hg->hdgt', kgp, eye).reshape(DP, HEADS * NT)
    vg = jnp.einsum('htd,hg->htgd', vgp, eye).reshape(HEADS * NT, DP)

    sc_gather, sc_scatter = _sc_kernels()
    x_perm = sc_gather(x, idx3)
    o = _attn(x_perm, x_perm, wqkv, wp, kg, vg)
    y = sc_scatter(o, idx3)
    return y[None]


# speedup vs baseline: 1.5241x; 1.5241x over previous
import functools

import jax
import jax.numpy as jnp
from jax import lax
from jax.experimental import pallas as pl
from jax.experimental.pallas import tpu as pltpu
from jax.experimental.pallas import tpu_sc as plsc

N = 50176
D = 96
HEADS = 4
HD = 24
HP = 32
DP = HEADS * HP
GS = 128
NG = N // GS
CG = 8
NCHUNK = NG // CG
NT = 8

NC = 2
NS = 16
NW = NC * NS
RPW = N // NW
CCH = 112
KCH = RPW // CCH

@functools.cache
def _sc_kernels():
    mesh = plsc.VectorSubcoreMesh(core_axis_name="c", subcore_axis_name="s")
    common = dict(
        mesh=mesh,
        compiler_params=pltpu.CompilerParams(use_tc_tiling_on_sc=False),
        out_type=jax.ShapeDtypeStruct((N, D), jnp.float32),
        scratch_types=[
            pltpu.VMEM((KCH, CCH), jnp.int32),
            pltpu.VMEM((2, CCH, D), jnp.float32),
            pltpu.SemaphoreType.DMA,
            pltpu.SemaphoreType.DMA,
        ],
    )


    @functools.partial(pl.kernel, **common)
    def sc_gather(x_hbm, idx_hbm, out_hbm, idx_v, rows_v, gsem, wsem):
        wid = lax.axis_index("s") * NC + lax.axis_index("c")
        base = wid * RPW
        pltpu.sync_copy(idx_hbm.at[wid], idx_v)
        gh = [None] * KCH
        wh = [None] * KCH
        for j in range(KCH):
            if j >= 2:
                wh[j - 2].wait()
            gh[j] = pltpu.async_copy(
                x_hbm.at[idx_v.at[j]], rows_v.at[j % 2], gsem)
            if j >= 1:
                wh[j - 1] = pltpu.async_copy(
                    rows_v.at[(j - 1) % 2],
                    out_hbm.at[pl.ds(base + (j - 1) * CCH, CCH)], wsem)
            gh[j].wait()
        wh[KCH - 2].wait()
        pltpu.async_copy(
            rows_v.at[(KCH - 1) % 2],
            out_hbm.at[pl.ds(base + (KCH - 1) * CCH, CCH)], wsem).wait()

    @functools.partial(pl.kernel, **common)
    def sc_scatter(o_hbm, idx_hbm, out_hbm, idx_v, rows_v, gsem, wsem):
        wid = lax.axis_index("s") * NC + lax.axis_index("c")
        base = wid * RPW
        pltpu.sync_copy(idx_hbm.at[wid], idx_v)
        lh = [None] * KCH
        sh = [None] * KCH
        for j in range(KCH):
            if j >= 2:
                sh[j - 2].wait()
            lh[j] = pltpu.async_copy(
                o_hbm.at[pl.ds(base + j * CCH, CCH)], rows_v.at[j % 2], gsem)
            if j >= 1:
                sh[j - 1] = pltpu.async_copy(
                    rows_v.at[(j - 1) % 2], out_hbm.at[idx_v.at[j - 1]], wsem)
            lh[j].wait()
        sh[KCH - 2].wait()
        pltpu.async_copy(
            rows_v.at[(KCH - 1) % 2],
            out_hbm.at[idx_v.at[KCH - 1]], wsem).wait()

    return sc_gather, sc_scatter


def _attn_body(xa_ref, xb_ref, wqkv_ref, wp_ref, kg_ref, vg_ref, ones_ref,
               out_ref):
    f32 = jnp.float32
    xa = xa_ref[...]
    xb = xb_ref[...]
    wqkv = wqkv_ref[...]
    wp = wp_ref[...]
    kg = kg_ref[...]
    vg = vg_ref[...]
    ones_bd = ones_ref[...]

    def dot(a, b, dn):
        return lax.dot_general(a, b, dn, preferred_element_type=f32)

    M = CG * GS
    xc = jnp.concatenate([xa, xb], axis=0)
    qkv = dot(xc, wqkv, (((1,), (0,)), ((), ())))
    q = qkv[:M, :DP]
    kf = qkv[:, DP:2 * DP]
    vf = qkv[:, 2 * DP:]

    sg = dot(q, kg, (((1,), (0,)), ((), ())))
    pg = jnp.exp(sg)
    lg = dot(pg, ones_bd, (((1,), (0,)), ((), ())))
    o2 = dot(pg, vg, (((1,), (0,)), ((), ()))) / lg

    o_parts = []
    for h in range(HEADS):
        sl = slice(h * HP, (h + 1) * HP)
        qh = q[:, sl].reshape(CG, GS, HP)
        kh = jnp.concatenate(
            [kf[:M, sl].reshape(CG, GS, HP),
             kf[GS:, sl].reshape(CG, GS, HP)], axis=1)
        vh = jnp.concatenate(
            [vf[:M, sl].reshape(CG, GS, HP),
             vf[GS:, sl].reshape(CG, GS, HP)], axis=1)
        p = jnp.exp(dot(qh, kh, (((2,), (2,)), ((0,), (0,)))))
        l = jnp.sum(p, axis=-1, keepdims=True)
        o1 = dot(p, vh, (((2,), (1,)), ((0,), (0,)))) / l
        o_parts.append(o1.reshape(M, HP))
    o1c = jnp.concatenate(o_parts, axis=1)
    out_ref[...] = dot(o1c + o2, wp, (((1,), (0,)), ((), ())))


_attn = pl.pallas_call(
    _attn_body,
    grid=(NCHUNK,),
    in_specs=[
        pl.BlockSpec((CG * GS, D), lambda c: (c, 0)),
        pl.BlockSpec((GS, D), lambda c: (jnp.minimum(CG * c + CG, NG - 1), 0)),
        pl.BlockSpec((D, 3 * DP), lambda c: (0, 0)),
        pl.BlockSpec((DP, D), lambda c: (0, 0)),
        pl.BlockSpec((DP, HEADS * NT), lambda c: (0, 0)),
        pl.BlockSpec((HEADS * NT, DP), lambda c: (0, 0)),
        pl.BlockSpec((HEADS * NT, DP), lambda c: (0, 0)),
    ],
    out_specs=pl.BlockSpec((CG * GS, D), lambda c: (c, 0)),
    out_shape=jax.ShapeDtypeStruct((N, D), jnp.float32),
)


def _pad_heads_rows(w):
    return jnp.pad(w.reshape(HEADS, HD, D),
                   ((0, 0), (0, HP - HD), (0, 0))).reshape(DP, D)


def kernel(normed_x, idx_last, k_global, v_global, Wq, Wk, Wv, Wproj):
    x = normed_x[0]
    perm = idx_last[0, :, 0].astype(jnp.int32)
    idx3 = perm.reshape(NW, KCH, CCH)

    scale = jnp.float32(HD ** -0.5)
    wqkv = jnp.concatenate(
        [_pad_heads_rows(Wq).T * scale, _pad_heads_rows(Wk).T,
         _pad_heads_rows(Wv).T], axis=1)
    wp = _pad_heads_rows(Wproj.T)
    eye = jnp.eye(HEADS, dtype=jnp.float32)
    kgp = jnp.pad(k_global, ((0, 0), (0, 0), (0, HP - HD)))
    vgp = jnp.pad(v_global, ((0, 0), (0, 0), (0, HP - HD)))
    kg = jnp.einsum('htd,hg->hdgt', kgp, eye).reshape(DP, HEADS * NT)
    vg = jnp.einsum('htd,hg->htgd', vgp, eye).reshape(HEADS * NT, DP)
    ones_bd = jnp.kron(eye, jnp.ones((NT, HP), dtype=jnp.float32))

    sc_gather, sc_scatter = _sc_kernels()
    x_perm = sc_gather(x, idx3)
    o = _attn(x_perm, x_perm, wqkv, wp, kg, vg, ones_bd)
    y = sc_scatter(o, idx3)
    return y[None]
